# baseline (speedup 1.0000x reference)
import functools

import jax
import jax.numpy as jnp
from jax import lax
from jax.experimental import pallas as pl
from jax.experimental.pallas import tpu as pltpu
from jax.experimental.pallas import tpu_sc as plsc

NCLS, NREG, NEXC = 1000, 100, 100
B = 16384
TBL = NCLS * NREG * NEXC

NC, NS, L = 2, 16, 16
NW = NC * NS
BPW = B // NW
CHUNK = 128
NCHUNK = BPW // CHUNK


def _body(p_hbm, r_hbm, e_hbm, el_hbm, eh_hbm,
          ss_hbm, dl_hbm, wl_hbm, wh_hbm,
          score_hbm, d_hbm, wlo_hbm, who_hbm, eho_hbm,
          base_hbm, conc_hbm, res_hbm,
          p_v, r_v, e_v, el_v, ehx_v, idx_v,
          ss_v, dl_v, wl_v, wh_v,
          d_b, eh_b, base_b, conc_b, res_b,
          sem):
    wid = lax.axis_index("s") * NC + lax.axis_index("c")
    base = wid * BPW

    pltpu.sync_copy(p_hbm.at[pl.ds(base, BPW)], p_v)
    pltpu.sync_copy(r_hbm.at[pl.ds(base, BPW)], r_v)
    pltpu.sync_copy(e_hbm.at[pl.ds(base, BPW)], e_v)
    pltpu.sync_copy(el_hbm.at[pl.ds(base, BPW)], el_v)
    pltpu.sync_copy(eh_hbm.at[pl.ds(base, BPW)], ehx_v)

    for i in range(BPW // L):
        sl = pl.ds(i * L, L)
        flat = p_v[sl] * (NREG * NEXC) + r_v[sl] * NEXC + e_v[sl]
        idx_v[i // (CHUNK // L), pl.ds((i % (CHUNK // L)) * L, L)] = flat

    copies = []
    for c in range(NCHUNK):
        dsl = pl.ds(c * CHUNK, CHUNK)
        copies.append(pltpu.async_copy(ss_hbm.at[idx_v.at[c]], ss_v.at[dsl], sem))
        copies.append(pltpu.async_copy(dl_hbm.at[idx_v.at[c]], dl_v.at[dsl], sem))
        copies.append(pltpu.async_copy(wl_hbm.at[idx_v.at[c]], wl_v.at[dsl], sem))
        copies.append(pltpu.async_copy(wh_hbm.at[idx_v.at[c]], wh_v.at[dsl], sem))
    for cp in copies:
        cp.wait()

    for i in range(BPW // L):
        sl = pl.ds(i * L, L)
        score = ss_v[sl]
        d = jnp.minimum(jnp.maximum(dl_v[sl], -2.0), 2.0)
        wl = wl_v[sl]
        wh = wh_v[sl]
        el = el_v[sl]
        e2x = jnp.exp(ehx_v[sl] * 2.0)
        eh = 1.0 - 2.0 / (e2x + 1.0)
        base_t = score + d
        conc_t = wl * el + wh * eh
        res = base_t * conc_t
        d_b[sl] = d
        eh_b[sl] = eh
        base_b[sl] = base_t
        conc_b[sl] = conc_t
        res_b[sl] = res

    out_sl = pl.ds(base, BPW)
    pltpu.sync_copy(ss_v, score_hbm.at[out_sl])
    pltpu.sync_copy(d_b, d_hbm.at[out_sl])
    pltpu.sync_copy(wl_v, wlo_hbm.at[out_sl])
    pltpu.sync_copy(wh_v, who_hbm.at[out_sl])
    pltpu.sync_copy(eh_b, eho_hbm.at[out_sl])
    pltpu.sync_copy(base_b, base_hbm.at[out_sl])
    pltpu.sync_copy(conc_b, conc_hbm.at[out_sl])
    pltpu.sync_copy(res_b, res_hbm.at[out_sl])


@jax.jit
def _run(p_idx, r_idx, e_idx, e_low_norm, e_high_norm, ss, dl, wl, wh):
    f32 = jnp.float32
    out = jax.ShapeDtypeStruct((B,), f32)
    call = pl.kernel(
        _body,
        mesh=plsc.VectorSubcoreMesh(core_axis_name="c", subcore_axis_name="s"),
        out_type=[out] * 8,
        scratch_types=[
            pltpu.VMEM((BPW,), jnp.int32),
            pltpu.VMEM((BPW,), jnp.int32),
            pltpu.VMEM((BPW,), jnp.int32),
            pltpu.VMEM((BPW,), f32),
            pltpu.VMEM((BPW,), f32),
            pltpu.VMEM((NCHUNK, CHUNK), jnp.int32),
            pltpu.VMEM((BPW,), f32),
            pltpu.VMEM((BPW,), f32),
            pltpu.VMEM((BPW,), f32),
            pltpu.VMEM((BPW,), f32),
            pltpu.VMEM((BPW,), f32),
            pltpu.VMEM((BPW,), f32),
            pltpu.VMEM((BPW,), f32),
            pltpu.VMEM((BPW,), f32),
            pltpu.VMEM((BPW,), f32),
            pltpu.SemaphoreType.DMA,
        ],
    )
    return call(p_idx, r_idx, e_idx, e_low_norm, e_high_norm,
                ss.reshape(TBL), dl.reshape(TBL),
                wl.reshape(TBL), wh.reshape(TBL))


def kernel(p_idx, r_idx, e_idx, e_low_norm, e_high_norm,
           static_scores, delta, w_L, w_H):
    score, d, wl, wh, eh, base_t, conc_t, res = _run(
        p_idx, r_idx, e_idx, e_low_norm, e_high_norm,
        static_scores, delta, w_L, w_H)
    el = e_low_norm.reshape(-1)
    return (res[:, None], score, d, wl, wh, el, eh, base_t, conc_t, res)


# speedup vs baseline: 1.5183x; 1.5183x over previous
import functools

import jax
import jax.numpy as jnp
from jax import lax
from jax.experimental import pallas as pl
from jax.experimental.pallas import tpu as pltpu
from jax.experimental.pallas import tpu_sc as plsc

NCLS, NREG, NEXC = 1000, 100, 100
B = 16384
TBL = NCLS * NREG * NEXC

NC, NS, L = 2, 16, 16
NW = NC * NS
BPW = B // NW
CHUNK = 128
NCHUNK = BPW // CHUNK


def _body(p_hbm, r_hbm, e_hbm, el_hbm, eh_hbm,
          ss_hbm, dl_hbm, wl_hbm, wh_hbm,
          score_hbm, d_hbm, wlo_hbm, who_hbm, eho_hbm,
          base_hbm, conc_hbm, res_hbm,
          p_v, r_v, e_v, el_v, ehx_v, idx_v,
          ss_v, dl_v, wl_v, wh_v,
          d_b, eh_b, base_b, conc_b, res_b,
          sem):
    wid = lax.axis_index("s") * NC + lax.axis_index("c")
    base = wid * BPW

    pltpu.sync_copy(p_hbm.at[pl.ds(base, BPW)], p_v)
    pltpu.sync_copy(r_hbm.at[pl.ds(base, BPW)], r_v)
    pltpu.sync_copy(e_hbm.at[pl.ds(base, BPW)], e_v)
    pltpu.sync_copy(el_hbm.at[pl.ds(base, BPW)], el_v)
    pltpu.sync_copy(eh_hbm.at[pl.ds(base, BPW)], ehx_v)

    for i in range(BPW // L):
        sl = pl.ds(i * L, L)
        flat = p_v[sl] * (NREG * NEXC) + r_v[sl] * NEXC + e_v[sl]
        idx_v[i // (CHUNK // L), pl.ds((i % (CHUNK // L)) * L, L)] = flat

    copies = []
    for c in range(NCHUNK):
        dsl = pl.ds(c * CHUNK, CHUNK)
        copies.append(pltpu.async_copy(ss_hbm.at[idx_v.at[c]], ss_v.at[dsl], sem))
        copies.append(pltpu.async_copy(dl_hbm.at[idx_v.at[c]], dl_v.at[dsl], sem))
        copies.append(pltpu.async_copy(wl_hbm.at[idx_v.at[c]], wl_v.at[dsl], sem))
        copies.append(pltpu.async_copy(wh_hbm.at[idx_v.at[c]], wh_v.at[dsl], sem))
    for cp in copies:
        cp.wait()

    for i in range(BPW // L):
        sl = pl.ds(i * L, L)
        score = ss_v[sl]
        d = jnp.minimum(jnp.maximum(dl_v[sl], -2.0), 2.0)
        wl = wl_v[sl]
        wh = wh_v[sl]
        el = el_v[sl]
        e2x = jnp.exp(ehx_v[sl] * 2.0)
        eh = 1.0 - 2.0 / (e2x + 1.0)
        base_t = score + d
        conc_t = wl * el + wh * eh
        res = base_t * conc_t
        d_b[sl] = d
        eh_b[sl] = eh
        base_b[sl] = base_t
        conc_b[sl] = conc_t
        res_b[sl] = res

    out_sl = pl.ds(base, BPW)
    pltpu.sync_copy(ss_v, score_hbm.at[out_sl])
    pltpu.sync_copy(d_b, d_hbm.at[out_sl])
    pltpu.sync_copy(wl_v, wlo_hbm.at[out_sl])
    pltpu.sync_copy(wh_v, who_hbm.at[out_sl])
    pltpu.sync_copy(eh_b, eho_hbm.at[out_sl])
    pltpu.sync_copy(base_b, base_hbm.at[out_sl])
    pltpu.sync_copy(conc_b, conc_hbm.at[out_sl])
    pltpu.sync_copy(res_b, res_hbm.at[out_sl])


@jax.jit
def _run(p_idx, r_idx, e_idx, e_low_norm, e_high_norm, ss, dl, wl, wh):
    f32 = jnp.float32
    out = jax.ShapeDtypeStruct((B,), f32)
    call = pl.kernel(
        _body,
        mesh=plsc.VectorSubcoreMesh(core_axis_name="c", subcore_axis_name="s"),
        out_type=[out] * 8,
        scratch_types=[
            pltpu.VMEM((BPW,), jnp.int32),
            pltpu.VMEM((BPW,), jnp.int32),
            pltpu.VMEM((BPW,), jnp.int32),
            pltpu.VMEM((BPW,), f32),
            pltpu.VMEM((BPW,), f32),
            pltpu.VMEM((NCHUNK, CHUNK), jnp.int32),
            pltpu.VMEM((BPW,), f32),
            pltpu.VMEM((BPW,), f32),
            pltpu.VMEM((BPW,), f32),
            pltpu.VMEM((BPW,), f32),
            pltpu.VMEM((BPW,), f32),
            pltpu.VMEM((BPW,), f32),
            pltpu.VMEM((BPW,), f32),
            pltpu.VMEM((BPW,), f32),
            pltpu.VMEM((BPW,), f32),
            pltpu.SemaphoreType.DMA,
        ],
    )
    one = jnp.where(p_idx[0] >= jnp.int32(-1), f32(1.0), f32(2.0))
    flat = lambda x: (x * one).reshape(TBL)
    return call(p_idx, r_idx, e_idx, e_low_norm, e_high_norm,
                flat(ss), flat(dl), flat(wl), flat(wh))


def kernel(p_idx, r_idx, e_idx, e_low_norm, e_high_norm,
           static_scores, delta, w_L, w_H):
    score, d, wl, wh, eh, base_t, conc_t, res = _run(
        p_idx, r_idx, e_idx, e_low_norm, e_high_norm,
        static_scores, delta, w_L, w_H)
    el = e_low_norm.reshape(-1)
    return (res[:, None], score, d, wl, wh, el, eh, base_t, conc_t, res)


# speedup vs baseline: 2.0730x; 1.3653x over previous
import functools

import jax
import jax.numpy as jnp
from jax import lax
from jax.experimental import pallas as pl
from jax.experimental.pallas import tpu as pltpu
from jax.experimental.pallas import tpu_sc as plsc

NCLS, NREG, NEXC = 1000, 100, 100
B = 16384
TBL = NCLS * NREG * NEXC

NC, NS, L = 2, 16, 16
NW = NC * NS
BPW = B // NW
CHUNK = 128
NCHUNK = BPW // CHUNK


def _body(p_hbm, r_hbm, e_hbm, el_hbm, eh_hbm,
          ss_hbm, dl_hbm, wl_hbm, wh_hbm,
          score_hbm, d_hbm, wlo_hbm, who_hbm, eho_hbm,
          base_hbm, conc_hbm, res_hbm,
          p_v, r_v, e_v, el_v, ehx_v, idx_v,
          ss_v, dl_v, wl_v, wh_v,
          d_b, eh_b, base_b, conc_b, res_b,
          sem):
    wid = lax.axis_index("s") * NC + lax.axis_index("c")
    base = wid * BPW

    pltpu.sync_copy(p_hbm.at[pl.ds(base, BPW)], p_v)
    pltpu.sync_copy(r_hbm.at[pl.ds(base, BPW)], r_v)
    pltpu.sync_copy(e_hbm.at[pl.ds(base, BPW)], e_v)
    pltpu.sync_copy(el_hbm.at[pl.ds(base, BPW)], el_v)
    pltpu.sync_copy(eh_hbm.at[pl.ds(base, BPW)], ehx_v)

    for i in range(BPW // L):
        sl = pl.ds(i * L, L)
        flat = (r_v[sl] * NEXC + e_v[sl]) * NCLS + p_v[sl]
        idx_v[i // (CHUNK // L), pl.ds((i % (CHUNK // L)) * L, L)] = flat

    copies = []
    for c in range(NCHUNK):
        dsl = pl.ds(c * CHUNK, CHUNK)
        copies.append(pltpu.async_copy(ss_hbm.at[idx_v.at[c]], ss_v.at[dsl], sem))
        copies.append(pltpu.async_copy(dl_hbm.at[idx_v.at[c]], dl_v.at[dsl], sem))
        copies.append(pltpu.async_copy(wl_hbm.at[idx_v.at[c]], wl_v.at[dsl], sem))
        copies.append(pltpu.async_copy(wh_hbm.at[idx_v.at[c]], wh_v.at[dsl], sem))
    for cp in copies:
        cp.wait()

    for i in range(BPW // L):
        sl = pl.ds(i * L, L)
        score = ss_v[sl]
        d = jnp.minimum(jnp.maximum(dl_v[sl], -2.0), 2.0)
        wl = wl_v[sl]
        wh = wh_v[sl]
        el = el_v[sl]
        e2x = jnp.exp(ehx_v[sl] * 2.0)
        eh = 1.0 - 2.0 / (e2x + 1.0)
        base_t = score + d
        conc_t = wl * el + wh * eh
        res = base_t * conc_t
        d_b[sl] = d
        eh_b[sl] = eh
        base_b[sl] = base_t
        conc_b[sl] = conc_t
        res_b[sl] = res

    out_sl = pl.ds(base, BPW)
    pltpu.sync_copy(ss_v, score_hbm.at[out_sl])
    pltpu.sync_copy(d_b, d_hbm.at[out_sl])
    pltpu.sync_copy(wl_v, wlo_hbm.at[out_sl])
    pltpu.sync_copy(wh_v, who_hbm.at[out_sl])
    pltpu.sync_copy(eh_b, eho_hbm.at[out_sl])
    pltpu.sync_copy(base_b, base_hbm.at[out_sl])
    pltpu.sync_copy(conc_b, conc_hbm.at[out_sl])
    pltpu.sync_copy(res_b, res_hbm.at[out_sl])


@jax.jit
def _run(p_idx, r_idx, e_idx, e_low_norm, e_high_norm, ss, dl, wl, wh):
    f32 = jnp.float32
    out = jax.ShapeDtypeStruct((B,), f32)
    call = pl.kernel(
        _body,
        mesh=plsc.VectorSubcoreMesh(core_axis_name="c", subcore_axis_name="s"),
        out_type=[out] * 8,
        scratch_types=[
            pltpu.VMEM((BPW,), jnp.int32),
            pltpu.VMEM((BPW,), jnp.int32),
            pltpu.VMEM((BPW,), jnp.int32),
            pltpu.VMEM((BPW,), f32),
            pltpu.VMEM((BPW,), f32),
            pltpu.VMEM((NCHUNK, CHUNK), jnp.int32),
            pltpu.VMEM((BPW,), f32),
            pltpu.VMEM((BPW,), f32),
            pltpu.VMEM((BPW,), f32),
            pltpu.VMEM((BPW,), f32),
            pltpu.VMEM((BPW,), f32),
            pltpu.VMEM((BPW,), f32),
            pltpu.VMEM((BPW,), f32),
            pltpu.VMEM((BPW,), f32),
            pltpu.VMEM((BPW,), f32),
            pltpu.SemaphoreType.DMA,
        ],
    )
    one = jnp.where(p_idx[0] >= jnp.int32(-1), f32(1.0), f32(2.0))
    flat = lambda x: (jnp.transpose(x, (1, 2, 0)) * one).reshape(TBL)
    return call(p_idx, r_idx, e_idx, e_low_norm, e_high_norm,
                flat(ss), flat(dl), flat(wl), flat(wh))


def kernel(p_idx, r_idx, e_idx, e_low_norm, e_high_norm,
           static_scores, delta, w_L, w_H):
    score, d, wl, wh, eh, base_t, conc_t, res = _run(
        p_idx, r_idx, e_idx, e_low_norm, e_high_norm,
        static_scores, delta, w_L, w_H)
    el = e_low_norm.reshape(-1)
    return (res[:, None], score, d, wl, wh, el, eh, base_t, conc_t, res)


# speedup vs baseline: 3.1711x; 1.5297x over previous
import functools

import jax
import jax.numpy as jnp
from jax import lax
from jax.experimental import pallas as pl
from jax.experimental.pallas import tpu as pltpu
from jax.experimental.pallas import tpu_sc as plsc

NCLS, NREG, NEXC = 1000, 100, 100
B = 16384
EPAD, PPAD = 104, 1024
TBL = NREG * EPAD * PPAD

NC, NS, L = 2, 16, 16
NW = NC * NS
BPW = B // NW
CHUNK = 128
NCHUNK = BPW // CHUNK


def _body(p_hbm, r_hbm, e_hbm, el_hbm, eh_hbm,
          ss_hbm, dl_hbm, wl_hbm, wh_hbm,
          score_hbm, d_hbm, wlo_hbm, who_hbm, eho_hbm,
          base_hbm, conc_hbm, res_hbm,
          p_v, r_v, e_v, el_v, ehx_v, idx_v,
          ss_v, dl_v, wl_v, wh_v,
          d_b, eh_b, base_b, conc_b, res_b,
          sem):
    wid = lax.axis_index("s") * NC + lax.axis_index("c")
    base = wid * BPW

    pltpu.sync_copy(p_hbm.at[pl.ds(base, BPW)], p_v)
    pltpu.sync_copy(r_hbm.at[pl.ds(base, BPW)], r_v)
    pltpu.sync_copy(e_hbm.at[pl.ds(base, BPW)], e_v)
    pltpu.sync_copy(el_hbm.at[pl.ds(base, BPW)], el_v)
    pltpu.sync_copy(eh_hbm.at[pl.ds(base, BPW)], ehx_v)

    for i in range(BPW // L):
        sl = pl.ds(i * L, L)
        r = r_v[sl]
        e = e_v[sl]
        p = p_v[sl]
        flat = (r * (EPAD * PPAD) + (e >> 3) * (8 * PPAD) + (e & 7) * 128
                + (p >> 7) * 1024 + (p & 127))
        idx_v[i // (CHUNK // L), pl.ds((i % (CHUNK // L)) * L, L)] = flat

    copies = []
    for c in range(NCHUNK):
        dsl = pl.ds(c * CHUNK, CHUNK)
        copies.append(pltpu.async_copy(ss_hbm.at[idx_v.at[c]], ss_v.at[dsl], sem))
        copies.append(pltpu.async_copy(dl_hbm.at[idx_v.at[c]], dl_v.at[dsl], sem))
        copies.append(pltpu.async_copy(wl_hbm.at[idx_v.at[c]], wl_v.at[dsl], sem))
        copies.append(pltpu.async_copy(wh_hbm.at[idx_v.at[c]], wh_v.at[dsl], sem))
    for cp in copies:
        cp.wait()

    for i in range(BPW // L):
        sl = pl.ds(i * L, L)
        score = ss_v[sl]
        d = jnp.minimum(jnp.maximum(dl_v[sl], -2.0), 2.0)
        wl = wl_v[sl]
        wh = wh_v[sl]
        el = el_v[sl]
        e2x = jnp.exp(ehx_v[sl] * 2.0)
        eh = 1.0 - 2.0 / (e2x + 1.0)
        base_t = score + d
        conc_t = wl * el + wh * eh
        res = base_t * conc_t
        d_b[sl] = d
        eh_b[sl] = eh
        base_b[sl] = base_t
        conc_b[sl] = conc_t
        res_b[sl] = res

    out_sl = pl.ds(base, BPW)
    pltpu.sync_copy(ss_v, score_hbm.at[out_sl])
    pltpu.sync_copy(d_b, d_hbm.at[out_sl])
    pltpu.sync_copy(wl_v, wlo_hbm.at[out_sl])
    pltpu.sync_copy(wh_v, who_hbm.at[out_sl])
    pltpu.sync_copy(eh_b, eho_hbm.at[out_sl])
    pltpu.sync_copy(base_b, base_hbm.at[out_sl])
    pltpu.sync_copy(conc_b, conc_hbm.at[out_sl])
    pltpu.sync_copy(res_b, res_hbm.at[out_sl])


@jax.jit
def _run(p_idx, r_idx, e_idx, e_low_norm, e_high_norm, ss, dl, wl, wh):
    f32 = jnp.float32
    out = jax.ShapeDtypeStruct((B,), f32)
    call = pl.kernel(
        _body,
        mesh=plsc.VectorSubcoreMesh(core_axis_name="c", subcore_axis_name="s"),
        out_type=[out] * 8,
        scratch_types=[
            pltpu.VMEM((BPW,), jnp.int32),
            pltpu.VMEM((BPW,), jnp.int32),
            pltpu.VMEM((BPW,), jnp.int32),
            pltpu.VMEM((BPW,), f32),
            pltpu.VMEM((BPW,), f32),
            pltpu.VMEM((NCHUNK, CHUNK), jnp.int32),
            pltpu.VMEM((BPW,), f32),
            pltpu.VMEM((BPW,), f32),
            pltpu.VMEM((BPW,), f32),
            pltpu.VMEM((BPW,), f32),
            pltpu.VMEM((BPW,), f32),
            pltpu.VMEM((BPW,), f32),
            pltpu.VMEM((BPW,), f32),
            pltpu.VMEM((BPW,), f32),
            pltpu.VMEM((BPW,), f32),
            pltpu.SemaphoreType.DMA,
        ],
    )
    flat = lambda x: jnp.pad(
        jnp.transpose(x, (1, 2, 0)),
        ((0, 0), (0, EPAD - NEXC), (0, PPAD - NCLS)),
    ).reshape(TBL)
    return call(p_idx, r_idx, e_idx, e_low_norm, e_high_norm,
                flat(ss), flat(dl), flat(wl), flat(wh))


def kernel(p_idx, r_idx, e_idx, e_low_norm, e_high_norm,
           static_scores, delta, w_L, w_H):
    score, d, wl, wh, eh, base_t, conc_t, res = _run(
        p_idx, r_idx, e_idx, e_low_norm, e_high_norm,
        static_scores, delta, w_L, w_H)
    el = e_low_norm.reshape(-1)
    return (res[:, None], score, d, wl, wh, el, eh, base_t, conc_t, res)


# speedup vs baseline: 3.1969x; 1.0081x over previous
import functools

import jax
import jax.numpy as jnp
from jax import lax
from jax.experimental import pallas as pl
from jax.experimental.pallas import tpu as pltpu
from jax.experimental.pallas import tpu_sc as plsc

NCLS, NREG, NEXC = 1000, 100, 100
B = 16384
EPAD, PPAD = 104, 1024
TBL = NREG * EPAD * PPAD

NC, NS, L = 2, 16, 16
NW = NC * NS
BPW = B // NW
CHUNK = 128
NCHUNK = BPW // CHUNK


def _body(p_hbm, r_hbm, e_hbm, el_hbm, eh_hbm,
          ss_hbm, dl_hbm, wl_hbm, wh_hbm,
          score_hbm, d_hbm, wlo_hbm, who_hbm, eho_hbm,
          base_hbm, conc_hbm, res_hbm,
          p_v, r_v, e_v, el_v, ehx_v, idx_v,
          ss_v, dl_v, wl_v, wh_v,
          d_b, eh_b, base_b, conc_b, res_b,
          sem):
    wid = lax.axis_index("s") * NC + lax.axis_index("c")
    base = wid * BPW

    pltpu.sync_copy(p_hbm.at[pl.ds(base, BPW)], p_v)
    pltpu.sync_copy(r_hbm.at[pl.ds(base, BPW)], r_v)
    pltpu.sync_copy(e_hbm.at[pl.ds(base, BPW)], e_v)
    pltpu.sync_copy(el_hbm.at[pl.ds(base, BPW)], el_v)
    pltpu.sync_copy(eh_hbm.at[pl.ds(base, BPW)], ehx_v)

    for i in range(BPW // L):
        sl = pl.ds(i * L, L)
        flat = (r_v[sl] * EPAD + e_v[sl]) * PPAD + p_v[sl]
        idx_v[i // (CHUNK // L), pl.ds((i % (CHUNK // L)) * L, L)] = flat

    copies = []
    for c in range(NCHUNK):
        dsl = pl.ds(c * CHUNK, CHUNK)
        copies.append(pltpu.async_copy(ss_hbm.at[idx_v.at[c]], ss_v.at[dsl], sem))
        copies.append(pltpu.async_copy(dl_hbm.at[idx_v.at[c]], dl_v.at[dsl], sem))
        copies.append(pltpu.async_copy(wl_hbm.at[idx_v.at[c]], wl_v.at[dsl], sem))
        copies.append(pltpu.async_copy(wh_hbm.at[idx_v.at[c]], wh_v.at[dsl], sem))
    for cp in copies:
        cp.wait()

    for i in range(BPW // L):
        sl = pl.ds(i * L, L)
        score = ss_v[sl]
        d = jnp.minimum(jnp.maximum(dl_v[sl], -2.0), 2.0)
        wl = wl_v[sl]
        wh = wh_v[sl]
        el = el_v[sl]
        e2x = jnp.exp(ehx_v[sl] * 2.0)
        eh = 1.0 - 2.0 / (e2x + 1.0)
        base_t = score + d
        conc_t = wl * el + wh * eh
        res = base_t * conc_t
        d_b[sl] = d
        eh_b[sl] = eh
        base_b[sl] = base_t
        conc_b[sl] = conc_t
        res_b[sl] = res

    out_sl = pl.ds(base, BPW)
    pltpu.sync_copy(ss_v, score_hbm.at[out_sl])
    pltpu.sync_copy(d_b, d_hbm.at[out_sl])
    pltpu.sync_copy(wl_v, wlo_hbm.at[out_sl])
    pltpu.sync_copy(wh_v, who_hbm.at[out_sl])
    pltpu.sync_copy(eh_b, eho_hbm.at[out_sl])
    pltpu.sync_copy(base_b, base_hbm.at[out_sl])
    pltpu.sync_copy(conc_b, conc_hbm.at[out_sl])
    pltpu.sync_copy(res_b, res_hbm.at[out_sl])


@jax.jit
def _run(p_idx, r_idx, e_idx, e_low_norm, e_high_norm, ss, dl, wl, wh):
    f32 = jnp.float32
    out = jax.ShapeDtypeStruct((B,), f32)
    call = pl.kernel(
        _body,
        mesh=plsc.VectorSubcoreMesh(core_axis_name="c", subcore_axis_name="s"),
        out_type=[out] * 8,
        scratch_types=[
            pltpu.VMEM((BPW,), jnp.int32),
            pltpu.VMEM((BPW,), jnp.int32),
            pltpu.VMEM((BPW,), jnp.int32),
            pltpu.VMEM((BPW,), f32),
            pltpu.VMEM((BPW,), f32),
            pltpu.VMEM((NCHUNK, CHUNK), jnp.int32),
            pltpu.VMEM((BPW,), f32),
            pltpu.VMEM((BPW,), f32),
            pltpu.VMEM((BPW,), f32),
            pltpu.VMEM((BPW,), f32),
            pltpu.VMEM((BPW,), f32),
            pltpu.VMEM((BPW,), f32),
            pltpu.VMEM((BPW,), f32),
            pltpu.VMEM((BPW,), f32),
            pltpu.VMEM((BPW,), f32),
            pltpu.SemaphoreType.DMA,
        ],
    )
    flat = lambda x: jnp.pad(
        jnp.transpose(x, (1, 2, 0)),
        ((0, 0), (0, EPAD - NEXC), (0, PPAD - NCLS)),
    ).reshape(TBL)
    return call(p_idx, r_idx, e_idx, e_low_norm, e_high_norm,
                flat(ss), flat(dl), flat(wl), flat(wh))


def kernel(p_idx, r_idx, e_idx, e_low_norm, e_high_norm,
           static_scores, delta, w_L, w_H):
    score, d, wl, wh, eh, base_t, conc_t, res = _run(
        p_idx, r_idx, e_idx, e_low_norm, e_high_norm,
        static_scores, delta, w_L, w_H)
    el = e_low_norm.reshape(-1)
    return (res[:, None], score, d, wl, wh, el, eh, base_t, conc_t, res)


# speedup vs baseline: 5.9006x; 1.8457x over previous
import functools

import jax
import jax.numpy as jnp
from jax import lax
from jax.experimental import pallas as pl
from jax.experimental.pallas import tpu as pltpu
from jax.experimental.pallas import tpu_sc as plsc

NCLS, NREG, NEXC = 1000, 100, 100
B = 16384
EPAD, PPAD = 104, 1024
TBL = NREG * EPAD * PPAD

NC, NS, L = 2, 16, 16
NW = NC * NS
BPW = B // NW
CHUNK = 128
NCHUNK = BPW // CHUNK


def _body(p_hbm, r_hbm, e_hbm, el_hbm, eh_hbm,
          ss_hbm, dl_hbm, wl_hbm, wh_hbm,
          score_hbm, d_hbm, wlo_hbm, who_hbm, eho_hbm,
          base_hbm, conc_hbm, res_hbm,
          p_v, r_v, e_v, el_v, ehx_v, idx_v,
          ss_v, dl_v, wl_v, wh_v,
          d_b, eh_b, base_b, conc_b, res_b,
          sem):
    wid = lax.axis_index("s") * NC + lax.axis_index("c")
    base = wid * BPW

    pltpu.sync_copy(p_hbm.at[pl.ds(base, BPW)], p_v)
    pltpu.sync_copy(r_hbm.at[pl.ds(base, BPW)], r_v)
    pltpu.sync_copy(e_hbm.at[pl.ds(base, BPW)], e_v)
    pltpu.sync_copy(el_hbm.at[pl.ds(base, BPW)], el_v)
    pltpu.sync_copy(eh_hbm.at[pl.ds(base, BPW)], ehx_v)

    for i in range(BPW // L):
        sl = pl.ds(i * L, L)
        r = r_v[sl]
        e = e_v[sl]
        p = p_v[sl]
        flat = (r * (EPAD * PPAD) + (e >> 3) * 8192 + (p >> 7) * 1024
                + (e & 7) * 128 + (p & 127))
        idx_v[i // (CHUNK // L), pl.ds((i % (CHUNK // L)) * L, L)] = flat

    copies = []
    for c in range(NCHUNK):
        dsl = pl.ds(c * CHUNK, CHUNK)
        copies.append(pltpu.async_copy(ss_hbm.at[idx_v.at[c]], ss_v.at[dsl], sem))
        copies.append(pltpu.async_copy(dl_hbm.at[idx_v.at[c]], dl_v.at[dsl], sem))
        copies.append(pltpu.async_copy(wl_hbm.at[idx_v.at[c]], wl_v.at[dsl], sem))
        copies.append(pltpu.async_copy(wh_hbm.at[idx_v.at[c]], wh_v.at[dsl], sem))
    for cp in copies:
        cp.wait()

    for i in range(BPW // L):
        sl = pl.ds(i * L, L)
        score = ss_v[sl]
        d = jnp.minimum(jnp.maximum(dl_v[sl], -2.0), 2.0)
        wl = wl_v[sl]
        wh = wh_v[sl]
        el = el_v[sl]
        e2x = jnp.exp(ehx_v[sl] * 2.0)
        eh = 1.0 - 2.0 / (e2x + 1.0)
        base_t = score + d
        conc_t = wl * el + wh * eh
        res = base_t * conc_t
        d_b[sl] = d
        eh_b[sl] = eh
        base_b[sl] = base_t
        conc_b[sl] = conc_t
        res_b[sl] = res

    out_sl = pl.ds(base, BPW)
    pltpu.sync_copy(ss_v, score_hbm.at[out_sl])
    pltpu.sync_copy(d_b, d_hbm.at[out_sl])
    pltpu.sync_copy(wl_v, wlo_hbm.at[out_sl])
    pltpu.sync_copy(wh_v, who_hbm.at[out_sl])
    pltpu.sync_copy(eh_b, eho_hbm.at[out_sl])
    pltpu.sync_copy(base_b, base_hbm.at[out_sl])
    pltpu.sync_copy(conc_b, conc_hbm.at[out_sl])
    pltpu.sync_copy(res_b, res_hbm.at[out_sl])


@jax.jit
def _run(p_idx, r_idx, e_idx, e_low_norm, e_high_norm, ss, dl, wl, wh):
    f32 = jnp.float32
    out = jax.ShapeDtypeStruct((B,), f32)
    call = pl.kernel(
        _body,
        mesh=plsc.VectorSubcoreMesh(core_axis_name="c", subcore_axis_name="s"),
        out_type=[out] * 8,
        scratch_types=[
            pltpu.VMEM((BPW,), jnp.int32),
            pltpu.VMEM((BPW,), jnp.int32),
            pltpu.VMEM((BPW,), jnp.int32),
            pltpu.VMEM((BPW,), f32),
            pltpu.VMEM((BPW,), f32),
            pltpu.VMEM((NCHUNK, CHUNK), jnp.int32),
            pltpu.VMEM((BPW,), f32),
            pltpu.VMEM((BPW,), f32),
            pltpu.VMEM((BPW,), f32),
            pltpu.VMEM((BPW,), f32),
            pltpu.VMEM((BPW,), f32),
            pltpu.VMEM((BPW,), f32),
            pltpu.VMEM((BPW,), f32),
            pltpu.VMEM((BPW,), f32),
            pltpu.VMEM((BPW,), f32),
            pltpu.SemaphoreType.DMA,
        ],
    )
    def flat(x):
        tp = jnp.pad(
            jnp.transpose(x, (1, 2, 0)),
            ((0, 0), (0, EPAD - NEXC), (0, PPAD - NCLS)),
        )
        t5 = tp.reshape(NREG, EPAD // 8, 8, PPAD // 128, 128)
        return jnp.transpose(t5, (0, 1, 3, 2, 4)).reshape(TBL)
    return call(p_idx, r_idx, e_idx, e_low_norm, e_high_norm,
                flat(ss), flat(dl), flat(wl), flat(wh))


def kernel(p_idx, r_idx, e_idx, e_low_norm, e_high_norm,
           static_scores, delta, w_L, w_H):
    score, d, wl, wh, eh, base_t, conc_t, res = _run(
        p_idx, r_idx, e_idx, e_low_norm, e_high_norm,
        static_scores, delta, w_L, w_H)
    el = e_low_norm.reshape(-1)
    return (res[:, None], score, d, wl, wh, el, eh, base_t, conc_t, res)
